# baseline (speedup 1.0000x reference)
import functools

import jax
import jax.numpy as jnp
from jax import lax
from jax.experimental import pallas as pl
from jax.experimental.pallas import tpu as pltpu
from jax.experimental.pallas import tpu_sc as plsc

_PARENT = (1, 1, 20, 2, 20, 4, 5, 6, 20, 8, 9, 10, 0, 12, 13, 14, 0, 16,
           17, 18, 1, 7, 7, 11, 11)

_J, _T = 25, 300
_N = 3072
_NW = 32
_PPW = _N // _NW
_HALF = _PPW // 2

_CHUNK_OFFS = tuple(range(0, _T - 16, 16)) + (_T - 16,)


def _plane_compute(inb, outb):
    for off in _CHUNK_OFFS:
        rows = [inb[j, pl.ds(off, 16)] for j in range(_J)]
        for j in range(_J):
            outb[j, pl.ds(off, 16)] = rows[j] - rows[_PARENT[j]]


@functools.partial(
    pl.kernel,
    mesh=plsc.VectorSubcoreMesh(core_axis_name="c", subcore_axis_name="s"),
    out_type=jax.ShapeDtypeStruct((_N, _J, _T), jnp.float32),
    scratch_types=[
        pltpu.VMEM((_J, _T), jnp.float32),
        pltpu.VMEM((_J, _T), jnp.float32),
        pltpu.VMEM((_J, _T), jnp.float32),
        pltpu.VMEM((_J, _T), jnp.float32),
        pltpu.SemaphoreType.DMA,
        pltpu.SemaphoreType.DMA,
        pltpu.SemaphoreType.DMA,
        pltpu.SemaphoreType.DMA,
    ],
    compiler_params=pltpu.CompilerParams(use_tc_tiling_on_sc=True),
)
def _sc_joint2bone(x_hbm, out_hbm, in0, in1, ot0, ot1,
                   isem0, isem1, osem0, osem1):
    wid = lax.axis_index("s") * 2 + lax.axis_index("c")
    base = wid * _PPW

    pltpu.async_copy(x_hbm.at[base], in0, isem0)

    def _wait_in(buf, sem):
        pltpu.make_async_copy(x_hbm.at[base], buf, sem).wait()

    def _wait_out(buf, sem):
        pltpu.make_async_copy(buf, out_hbm.at[base], sem).wait()

    def body(t, _):
        cur0 = base + 2 * t

        pltpu.async_copy(x_hbm.at[cur0 + 1], in1, isem1)
        _wait_in(in0, isem0)

        @pl.when(t > 0)
        def _():
            _wait_out(ot0, osem0)

        _plane_compute(in0, ot0)
        pltpu.async_copy(ot0, out_hbm.at[cur0], osem0)

        @pl.when(t < _HALF - 1)
        def _():
            pltpu.async_copy(x_hbm.at[cur0 + 2], in0, isem0)

        _wait_in(in1, isem1)

        @pl.when(t > 0)
        def _():
            _wait_out(ot1, osem1)

        _plane_compute(in1, ot1)
        pltpu.async_copy(ot1, out_hbm.at[cur0 + 1], osem1)
        return _

    lax.fori_loop(0, _HALF, body, None)
    _wait_out(ot0, osem0)
    _wait_out(ot1, osem1)


def kernel(joint):
    B, C, J, T = joint.shape
    assert (B * C, J, T) == (_N, _J, _T)
    x = joint.reshape(_N, J, T)
    out = _sc_joint2bone(x)
    return out.reshape(B, C, J, T)


# speedup vs baseline: 3.0930x; 3.0930x over previous
import functools

import jax
import jax.numpy as jnp
from jax import lax
from jax.experimental import pallas as pl
from jax.experimental.pallas import tpu as pltpu
from jax.experimental.pallas import tpu_sc as plsc

_PARENT = (1, 1, 20, 2, 20, 4, 5, 6, 20, 8, 9, 10, 0, 12, 13, 14, 0, 16,
           17, 18, 1, 7, 7, 11, 11)

_C, _J, _T, _B = 3, 25, 300, 1024
_NW = 32
_NTASK = _C * _T
_TPW = -(-_NTASK // _NW)


def _slab_compute(inb, outb):
    def col(k, _):
        off = k * 16
        rows = [inb[j, pl.ds(off, 16)] for j in range(_J)]
        for j in range(_J):
            outb[j, pl.ds(off, 16)] = rows[j] - rows[_PARENT[j]]
        return _

    lax.fori_loop(0, _B // 16, col, None)


@functools.partial(
    pl.kernel,
    mesh=plsc.VectorSubcoreMesh(core_axis_name="c", subcore_axis_name="s"),
    out_type=jax.ShapeDtypeStruct((_C, _J, _T, _B), jnp.float32),
    scratch_types=[
        pltpu.VMEM((_J, _B), jnp.float32),
        pltpu.VMEM((_J, _B), jnp.float32),
        pltpu.VMEM((_J, _B), jnp.float32),
        pltpu.VMEM((_J, _B), jnp.float32),
        pltpu.SemaphoreType.DMA,
        pltpu.SemaphoreType.DMA,
        pltpu.SemaphoreType.DMA,
        pltpu.SemaphoreType.DMA,
    ],
)
def _sc_joint2bone(x_hbm, out_hbm, in0, in1, ot0, ot1,
                   isem0, isem1, osem0, osem1):
    wid = lax.axis_index("s") * 2 + lax.axis_index("c")

    def task_ct(i):
        q = lax.min(wid + _NW * i, _NTASK - 1)
        return q // _T, q % _T

    def start_in(i, buf, sem):
        c, t = task_ct(i)
        pltpu.async_copy(x_hbm.at[c, :, t, :], buf, sem)

    def wait_in(buf, sem):
        pltpu.make_async_copy(x_hbm.at[0, :, 0, :], buf, sem).wait()

    def start_out(i, buf, sem):
        c, t = task_ct(i)
        pltpu.async_copy(buf, out_hbm.at[c, :, t, :], sem)

    def wait_out(buf, sem):
        pltpu.make_async_copy(buf, out_hbm.at[0, :, 0, :], sem).wait()

    start_in(0, in0, isem0)

    def body(u, _):
        ia = 2 * u
        start_in(ia + 1, in1, isem1)
        wait_in(in0, isem0)

        @pl.when(u > 0)
        def _():
            wait_out(ot0, osem0)

        _slab_compute(in0, ot0)
        start_out(ia, ot0, osem0)

        start_in(ia + 2, in0, isem0)
        wait_in(in1, isem1)

        @pl.when(u > 0)
        def _():
            wait_out(ot1, osem1)

        _slab_compute(in1, ot1)
        start_out(ia + 1, ot1, osem1)
        return _

    lax.fori_loop(0, (_TPW - 1) // 2, body, None)
    wait_in(in0, isem0)
    wait_out(ot0, osem0)
    _slab_compute(in0, ot0)
    start_out(_TPW - 1, ot0, osem0)
    wait_out(ot0, osem0)
    wait_out(ot1, osem1)


def kernel(joint):
    B, C, J, T = joint.shape
    assert (C, J, T, B) == (_C, _J, _T, _B)
    y = jnp.transpose(joint, (1, 2, 3, 0))
    out = _sc_joint2bone(y)
    return jnp.transpose(out, (3, 0, 1, 2))
